# baseline (speedup 1.0000x reference)
import functools
import jax
import jax.numpy as jnp
from jax import lax
from jax.experimental import pallas as pl
from jax.experimental.pallas import tpu as pltpu
from jax.experimental.pallas import tpu_sc as plsc

H = 128
_PH = jax.lax.Precision.HIGHEST
LOG2 = 0.6931471805599453

_NC = 2
_NS = 16
_NW = _NC * _NS


def _ssp(x):
    return jnp.maximum(x, 0.0) + jnp.log1p(jnp.exp(-jnp.abs(x))) - LOG2


def _ln(x, g, b):
    m = jnp.mean(x, axis=-1, keepdims=True)
    v = jnp.mean((x - m) ** 2, axis=-1, keepdims=True)
    return (x - m) * jax.lax.rsqrt(v + 1e-5) * g + b


def _pack_bf16(x):
    lo = x[:, :64].astype(jnp.bfloat16).astype(jnp.float32)
    hi = x[:, 64:].astype(jnp.bfloat16).astype(jnp.float32)
    lo_b = lax.bitcast_convert_type(lo, jnp.int32)
    hi_b = lax.bitcast_convert_type(hi, jnp.int32)
    return ((lo_b >> 16) & jnp.int32(0xFFFF)) | (hi_b & jnp.int32(-65536))


def _unpack_bf16(gi):
    lo = lax.bitcast_convert_type(gi << 16, jnp.float32)
    hi = lax.bitcast_convert_type(gi & jnp.int32(-65536), jnp.float32)
    return lo, hi


def _full(shape):
    n = len(shape)
    return pl.BlockSpec(shape, lambda i, _n=n: (0,) * _n)


def _sc_gather(table, idx):
    E = idx.shape[0]
    dtype = table.dtype
    W = table.shape[1]
    per_w = E // _NW
    C = 400 if per_w % 400 == 0 else 200
    n_chunks = per_w // C

    mesh = plsc.VectorSubcoreMesh(core_axis_name="c", subcore_axis_name="s")

    @functools.partial(
        pl.kernel,
        out_type=jax.ShapeDtypeStruct((E, W), dtype),
        mesh=mesh,
        scratch_types=[
            pltpu.VMEM((C,), jnp.int32),
            pltpu.VMEM((C, W), dtype),
            pltpu.SemaphoreType.DMA,
        ],
        compiler_params=pltpu.CompilerParams(use_tc_tiling_on_sc=False),
    )
    def gk(table_hbm, idx_hbm, out_hbm, idx_v, rows_v, sem):
        wid = lax.axis_index("s") * _NC + lax.axis_index("c")
        base = wid * per_w

        def body(i, carry):
            off = base + i * C
            pltpu.sync_copy(idx_hbm.at[pl.ds(off, C)], idx_v)
            pltpu.async_copy(table_hbm.at[idx_v], rows_v, sem).wait()
            pltpu.sync_copy(rows_v, out_hbm.at[pl.ds(off, C)])
            return carry

        lax.fori_loop(0, n_chunks, body, 0)

    return gk(table, idx)


def _sc_scatter(vals, idx, n_nodes, init=None):
    E = idx.shape[0]
    per_w = E // _NW
    C = 200
    n_chunks = per_w // C
    n_pad = ((n_nodes + _NS * 8 - 1) // (_NS * 8)) * (_NS * 8)
    rows_per_s = n_pad // _NS

    mesh = plsc.VectorSubcoreMesh(core_axis_name="c", subcore_axis_name="s")
    if init is None:
        init = jnp.zeros((_NC, n_pad, H), jnp.float32)

    @functools.partial(
        pl.kernel,
        out_type=jax.ShapeDtypeStruct((_NC, n_pad, H), jnp.float32),
        mesh=mesh,
        scratch_types=[
            pltpu.VMEM((C,), jnp.int32),
            pltpu.VMEM((C, H), jnp.float32),
            pltpu.VMEM_SHARED((n_pad, H), jnp.float32),
        ],
    )
    def sk(vals_hbm, idx_hbm, init_hbm, out_hbm, idx_v, vals_v, acc_sh):
        c = lax.axis_index("c")
        s = lax.axis_index("s")
        pltpu.sync_copy(
            init_hbm.at[c, pl.ds(s * rows_per_s, rows_per_s)],
            acc_sh.at[pl.ds(s * rows_per_s, rows_per_s)],
        )
        plsc.subcore_barrier()

        base = (s * _NC + c) * per_w

        def body(i, carry):
            off = base + i * C
            pltpu.sync_copy(idx_hbm.at[pl.ds(off, C)], idx_v)
            pltpu.sync_copy(vals_hbm.at[pl.ds(off, C)], vals_v)
            pltpu.sync_copy(vals_v, acc_sh.at[idx_v], add=True)
            return carry

        lax.fori_loop(0, n_chunks, body, 0)
        plsc.subcore_barrier()
        pltpu.sync_copy(
            acc_sh.at[pl.ds(s * rows_per_s, rows_per_s)],
            out_hbm.at[c, pl.ds(s * rows_per_s, rows_per_s)],
        )

    return sk(vals, idx, init)


_BN = 2000
_BE = 2000


def _enc_node_body(z_ref, emb_ref, W1, b1, W2, b2, emf, emb1, x_ref, xw_ref):
    zb = z_ref[0, 0, :]
    onehot = (zb[:, None] == lax.broadcasted_iota(jnp.int32, (1, 90), 1)).astype(
        jnp.float32
    )
    x0 = jnp.dot(onehot, emb_ref[...], preferred_element_type=jnp.float32,
                 precision=_PH)
    h = jnp.maximum(
        jnp.dot(x0, W1[...], preferred_element_type=jnp.float32, precision=_PH)
        + b1[...], 0.0)
    x = jnp.dot(h, W2[...], preferred_element_type=jnp.float32, precision=_PH) + b2[...]
    x_ref[...] = x
    xw_ref[...] = _pack_bf16(
        jnp.dot(x, emf[...], preferred_element_type=jnp.float32, precision=_PH)
        + emb1[...])


def _enc_node(z, emb, W1, b1, W2, b2, emf, emb1):
    n = z.shape[0]
    grid = n // _BN
    z3 = z.reshape(grid, 1, _BN)
    return pl.pallas_call(
        _enc_node_body,
        grid=(grid,),
        in_specs=[
            pl.BlockSpec((1, 1, _BN), lambda i: (i, 0, 0)),
            _full(emb.shape), _full(W1.shape), _full(b1.shape),
            _full(W2.shape), _full(b2.shape), _full(emf.shape), _full(emb1.shape),
        ],
        out_specs=[
            pl.BlockSpec((_BN, H), lambda i: (i, 0)),
            pl.BlockSpec((_BN, H // 2), lambda i: (i, 0)),
        ],
        out_shape=[
            jax.ShapeDtypeStruct((n, H), jnp.float32),
            jax.ShapeDtypeStruct((n, H // 2), jnp.int32),
        ],
    )(z3, emb, W1, b1, W2, b2, emf, emb1)


def _enc_edge_body(ea_ref, W1, b1, W2, b2, out_ref):
    h = jnp.maximum(
        jnp.dot(ea_ref[...], W1[...], preferred_element_type=jnp.float32) + b1[...], 0.0
    )
    out_ref[...] = jnp.dot(h, W2[...], preferred_element_type=jnp.float32) + b2[...]


def _enc_edge(edge_attr, W1, b1, W2, b2):
    e, nb = edge_attr.shape
    grid = e // _BE
    return pl.pallas_call(
        _enc_edge_body,
        grid=(grid,),
        in_specs=[
            pl.BlockSpec((_BE, nb), lambda i: (i, 0)),
            _full(W1.shape), _full(b1.shape), _full(W2.shape), _full(b2.shape),
        ],
        out_specs=pl.BlockSpec((_BE, H), lambda i: (i, 0)),
        out_shape=jax.ShapeDtypeStruct((e, H), jnp.float32),
    )(edge_attr, W1, b1, W2, b2)


def _edge_step_body(g_ref, ea_ref, W1c, W2, b2, new_ref, sum_ref):
    ea = ea_ref[...]
    lo, hi = _unpack_bf16(g_ref[...])
    m = jnp.dot(ea, W1c[...], preferred_element_type=jnp.float32)
    t = jnp.concatenate([_ssp(lo + m[:, :64]), _ssp(hi + m[:, 64:])], axis=1)
    ea_new = jnp.dot(t, W2[...], preferred_element_type=jnp.float32) + b2[...]
    new_ref[...] = ea_new
    sum_ref[...] = ea + ea_new


def _edge_step(gathered, ea, W1c, W2, b2):
    e = ea.shape[0]
    grid = e // _BE
    return pl.pallas_call(
        _edge_step_body,
        grid=(grid,),
        in_specs=[
            pl.BlockSpec((_BE, H // 2), lambda i: (i, 0)),
            pl.BlockSpec((_BE, H), lambda i: (i, 0)),
            _full(W1c.shape), _full(W2.shape), _full(b2.shape),
        ],
        out_specs=[
            pl.BlockSpec((_BE, H), lambda i: (i, 0)),
            pl.BlockSpec((_BE, H), lambda i: (i, 0)),
        ],
        out_shape=[
            jax.ShapeDtypeStruct((e, H), jnp.float32),
            jax.ShapeDtypeStruct((e, H), jnp.float32),
        ],
    )(gathered, ea, W1c, W2, b2)


def _node_step_body(x_ref, a0_ref, a1_ref, W1a, W1b, b1, W2, b2, emf, emb1,
                    x_out_ref, xw_ref):
    x = x_ref[...]
    agg = a0_ref[0, :, :] + a1_ref[0, :, :]
    t = _ssp(
        jnp.dot(x, W1a[...], preferred_element_type=jnp.float32, precision=_PH)
        + jnp.dot(agg, W1b[...], preferred_element_type=jnp.float32, precision=_PH)
        + b1[...]
    )
    x_out = (x + jnp.dot(t, W2[...], preferred_element_type=jnp.float32,
                         precision=_PH) + b2[...])
    x_out_ref[...] = x_out
    xw_ref[...] = _pack_bf16(
        jnp.dot(x_out, emf[...], preferred_element_type=jnp.float32, precision=_PH)
        + emb1[...])


def _node_step(x, agg2, W1a, W1b, b1, W2, b2, emf, emb1):
    n = x.shape[0]
    grid = n // _BN
    return pl.pallas_call(
        _node_step_body,
        grid=(grid,),
        in_specs=[
            pl.BlockSpec((_BN, H), lambda i: (i, 0)),
            pl.BlockSpec((1, _BN, H), lambda i: (0, i, 0)),
            pl.BlockSpec((1, _BN, H), lambda i: (1, i, 0)),
            _full(W1a.shape), _full(W1b.shape), _full(b1.shape),
            _full(W2.shape), _full(b2.shape), _full(emf.shape), _full(emb1.shape),
        ],
        out_specs=[
            pl.BlockSpec((_BN, H), lambda i: (i, 0)),
            pl.BlockSpec((_BN, H // 2), lambda i: (i, 0)),
        ],
        out_shape=[
            jax.ShapeDtypeStruct((n, H), jnp.float32),
            jax.ShapeDtypeStruct((n, H // 2), jnp.int32),
        ],
    )(x, agg2, agg2, W1a, W1b, b1, W2, b2, emf, emb1)


def _dec_body(x1, x2, x3, xe, batch_ref, pmW, pmb, pmg, pmbt,
              qmW, qmb, qmg, qmbt, recW, recb,
              recon_ref, pp_ref, pe_ref):
    i = pl.program_id(0)
    bat = batch_ref[0, 0, :]
    onehot = (bat[:, None] == lax.broadcasted_iota(jnp.int32, (1, 64), 1)).astype(
        jnp.float32
    )

    @pl.when(i == 0)
    def _():
        pp_ref[...] = jnp.zeros_like(pp_ref)
        pe_ref[...] = jnp.zeros_like(pe_ref)

    for k, xr in enumerate((x1, x2, x3)):
        xv = xr[...]
        recon_ref[k, :, :] = (
            jnp.dot(xv, recW[...], preferred_element_type=jnp.float32) + recb[...]
        )
        h = xv
        for l in range(3):
            h = _ssp(_ln(
                jnp.dot(h, pmW[l], preferred_element_type=jnp.float32) + pmb[l],
                pmg[l], pmbt[l]))
        pp_ref[k, :, :] += jnp.dot(onehot.T, h, preferred_element_type=jnp.float32)

    he = xe[...]
    for l in range(3):
        he = _ssp(_ln(
            jnp.dot(he, qmW[l], preferred_element_type=jnp.float32) + qmb[l],
            qmg[l], qmbt[l]))
    pe_ref[...] += jnp.dot(onehot.T, he, preferred_element_type=jnp.float32)


def _decoder(x1, x2, x3, enc_x, batch, pmW, pmb, pmg, pmbt, qmW, qmb, qmg, qmbt,
             recW, recb):
    n = batch.shape[0]
    grid = n // _BN
    b3 = batch.reshape(grid, 1, _BN)
    return pl.pallas_call(
        _dec_body,
        grid=(grid,),
        in_specs=[
            pl.BlockSpec((_BN, H), lambda i: (i, 0)),
            pl.BlockSpec((_BN, H), lambda i: (i, 0)),
            pl.BlockSpec((_BN, H), lambda i: (i, 0)),
            pl.BlockSpec((_BN, H), lambda i: (i, 0)),
            pl.BlockSpec((1, 1, _BN), lambda i: (i, 0, 0)),
            _full(pmW.shape), _full(pmb.shape), _full(pmg.shape), _full(pmbt.shape),
            _full(qmW.shape), _full(qmb.shape), _full(qmg.shape), _full(qmbt.shape),
            _full(recW.shape), _full(recb.shape),
        ],
        out_specs=[
            pl.BlockSpec((3, _BN, 3), lambda i: (0, i, 0)),
            pl.BlockSpec((3, 64, H), lambda i: (0, 0, 0)),
            pl.BlockSpec((64, H), lambda i: (0, 0)),
        ],
        out_shape=[
            jax.ShapeDtypeStruct((3, n, 3), jnp.float32),
            jax.ShapeDtypeStruct((3, 64, H), jnp.float32),
            jax.ShapeDtypeStruct((64, H), jnp.float32),
        ],
    )(x1, x2, x3, enc_x, b3, pmW, pmb, pmg, pmbt, qmW, qmb, qmg, qmbt, recW, recb)


def _target_body(pp_ref, pe_ref, pW, pb, eW, eb, out_ref):
    pe_term = jnp.dot(pe_ref[...], eW[...], preferred_element_type=jnp.float32) + eb[...]
    for k in range(3):
        tk = jnp.dot(pp_ref[k], pW[...], preferred_element_type=jnp.float32) + pb[...]
        out_ref[k, :] = (tk + pe_term)[:, 0]


def _target(pp, pe, pW, pb, eW, eb):
    return pl.pallas_call(
        _target_body,
        grid=(1,),
        in_specs=[
            _full(pp.shape), _full(pe.shape), _full(pW.shape),
            _full(pb.shape), _full(eW.shape), _full(eb.shape),
        ],
        out_specs=pl.BlockSpec((3, 64), lambda i: (0, 0)),
        out_shape=jax.ShapeDtypeStruct((3, 64), jnp.float32),
    )(pp, pe, pW, pb, eW, eb)


def kernel(z, edge_attr, edge_index, batch, params):
    p = params
    n_nodes = z.shape[0]

    em_W1 = p['em_W1']
    em_fold = em_W1[:H] + em_W1[H:2 * H]
    em_W1c = em_W1[2 * H:]
    nm_W1a = p['nm_W1'][:H]
    nm_W1b = p['nm_W1'][H:]

    row = edge_index[0]
    dst = edge_index[1]
    n_edges = row.shape[0]
    e2 = n_edges // 2
    row_h = (row[:e2], row[e2:])
    dst_h = (dst[:e2], dst[e2:])

    enc_x, xw = _enc_node(z, p['embed'], p['ne_W1'], p['ne_b1'], p['ne_W2'],
                          p['ne_b2'], em_fold, p['em_b1'])
    ea_h = [
        _enc_edge(edge_attr[:e2], p['ee_W1'], p['ee_b1'], p['ee_W2'], p['ee_b2']),
        _enc_edge(edge_attr[e2:], p['ee_W1'], p['ee_b1'], p['ee_W2'], p['ee_b2']),
    ]

    x = enc_x
    xs = []
    for _b in range(3):
        for _m in range(2):
            new_h = [None, None]
            part = None
            for h in range(2):
                g = _sc_gather(xw, row_h[h])
                ea_new, new_h[h] = _edge_step(g, ea_h[h], em_W1c,
                                              p['em_W2'], p['em_b2'])
                part = _sc_scatter(ea_new, dst_h[h], n_nodes, init=part)
            ea_h = new_h
            x, xw = _node_step(x, part, nm_W1a, nm_W1b, p['nm_b1'],
                               p['nm_W2'], p['nm_b2'], em_fold, p['em_b1'])
        xs.append(x)

    x_list = jnp.stack(xs, axis=0)

    pmW = jnp.stack(p['pm_W']); pmb = jnp.stack(p['pm_b'])
    pmg = jnp.stack(p['pm_g']); pmbt = jnp.stack(p['pm_beta'])
    qmW = jnp.stack(p['qm_W']); qmb = jnp.stack(p['qm_b'])
    qmg = jnp.stack(p['qm_g']); qmbt = jnp.stack(p['qm_beta'])

    recon, pooled_proc, pooled_enc = _decoder(
        xs[0], xs[1], xs[2], enc_x, batch,
        pmW, pmb, pmg, pmbt, qmW, qmb, qmg, qmbt, p['rec_W'], p['rec_b'])

    target = _target(pooled_proc, pooled_enc, p['pW'], p['pb'], p['eW'], p['eb'])

    return (target, recon, x_list)


# speedup vs baseline: 1.2957x; 1.2957x over previous
import functools
import jax
import jax.numpy as jnp
from jax import lax
from jax.experimental import pallas as pl
from jax.experimental.pallas import tpu as pltpu
from jax.experimental.pallas import tpu_sc as plsc

H = 128
LOG2 = 0.6931471805599453

_NC = 2
_NS = 16
_NW = _NC * _NS


def _ssp(x):
    return jnp.maximum(x, 0.0) + jnp.log1p(jnp.exp(-jnp.abs(x))) - LOG2


def _ln(x, g, b):
    m = jnp.mean(x, axis=-1, keepdims=True)
    v = jnp.mean((x - m) ** 2, axis=-1, keepdims=True)
    return (x - m) * jax.lax.rsqrt(v + 1e-5) * g + b


def _full(shape):
    n = len(shape)
    return pl.BlockSpec(shape, lambda i, _n=n: (0,) * _n)


def _sc_gather(table, idx):
    E = idx.shape[0]
    per_w = E // _NW
    C = 400 if per_w % 400 == 0 else 200
    n_chunks = per_w // C
    n_pairs = n_chunks // 2
    tail = n_chunks - 2 * n_pairs

    mesh = plsc.VectorSubcoreMesh(core_axis_name="c", subcore_axis_name="s")

    @functools.partial(
        pl.kernel,
        out_type=jax.ShapeDtypeStruct((E, H), jnp.float32),
        mesh=mesh,
        scratch_types=[
            pltpu.VMEM((C,), jnp.int32),
            pltpu.VMEM((C, H), jnp.float32),
            pltpu.VMEM((C, H), jnp.float32),
            pltpu.SemaphoreType.DMA,
            pltpu.SemaphoreType.DMA,
            pltpu.SemaphoreType.DMA,
        ],
    )
    def gk(table_hbm, idx_hbm, out_hbm, idx_v, rows0, rows1, semg, semw0, semw1):
        wid = lax.axis_index("s") * _NC + lax.axis_index("c")
        base = wid * per_w

        def gather_into(off, rows_v):
            pltpu.sync_copy(idx_hbm.at[pl.ds(off, C)], idx_v)
            pltpu.async_copy(table_hbm.at[idx_v], rows_v, semg).wait()

        def body(j, carry):
            off_a = base + (2 * j) * C
            off_b = off_a + C
            gather_into(off_a, rows0)

            @pl.when(j > 0)
            def _():
                pltpu.make_async_copy(
                    rows1, out_hbm.at[pl.ds(off_b - 2 * C, C)], semw1).wait()

            pltpu.async_copy(rows0, out_hbm.at[pl.ds(off_a, C)], semw0)
            gather_into(off_b, rows1)
            pltpu.make_async_copy(
                rows0, out_hbm.at[pl.ds(off_a, C)], semw0).wait()
            pltpu.async_copy(rows1, out_hbm.at[pl.ds(off_b, C)], semw1)
            return carry

        lax.fori_loop(0, n_pairs, body, 0)
        if n_pairs > 0:
            last_b = base + (2 * n_pairs - 1) * C
            pltpu.make_async_copy(
                rows1, out_hbm.at[pl.ds(last_b, C)], semw1).wait()
        if tail:
            off_t = base + 2 * n_pairs * C
            gather_into(off_t, rows0)
            pltpu.sync_copy(rows0, out_hbm.at[pl.ds(off_t, C)])

    return gk(table, idx)


def _sc_scatter(vals, idx, n_nodes, init=None):
    E = idx.shape[0]
    per_w = E // _NW
    C = 200
    n_chunks = per_w // C
    n_pad = ((n_nodes + _NS * 8 - 1) // (_NS * 8)) * (_NS * 8)
    rows_per_s = n_pad // _NS

    mesh = plsc.VectorSubcoreMesh(core_axis_name="c", subcore_axis_name="s")
    if init is None:
        init = jnp.zeros((_NC, n_pad, H), jnp.float32)

    @functools.partial(
        pl.kernel,
        out_type=jax.ShapeDtypeStruct((_NC, n_pad, H), jnp.float32),
        mesh=mesh,
        scratch_types=[
            pltpu.VMEM((C,), jnp.int32),
            pltpu.VMEM((C, H), jnp.float32),
            pltpu.VMEM_SHARED((n_pad, H), jnp.float32),
        ],
    )
    def sk(vals_hbm, idx_hbm, init_hbm, out_hbm, idx_v, vals_v, acc_sh):
        c = lax.axis_index("c")
        s = lax.axis_index("s")
        pltpu.sync_copy(
            init_hbm.at[c, pl.ds(s * rows_per_s, rows_per_s)],
            acc_sh.at[pl.ds(s * rows_per_s, rows_per_s)],
        )
        plsc.subcore_barrier()

        base = (s * _NC + c) * per_w

        def body(i, carry):
            off = base + i * C
            pltpu.sync_copy(idx_hbm.at[pl.ds(off, C)], idx_v)
            pltpu.sync_copy(vals_hbm.at[pl.ds(off, C)], vals_v)
            pltpu.sync_copy(vals_v, acc_sh.at[idx_v], add=True)
            return carry

        lax.fori_loop(0, n_chunks, body, 0)
        plsc.subcore_barrier()
        pltpu.sync_copy(
            acc_sh.at[pl.ds(s * rows_per_s, rows_per_s)],
            out_hbm.at[c, pl.ds(s * rows_per_s, rows_per_s)],
        )

    return sk(vals, idx, init)


_BN = 2000
_BE = 2000


def _enc_node_body(z_ref, emb_ref, W1, b1, W2, b2, emf, emb1, x_ref, xw_ref):
    zb = z_ref[0, 0, :]
    onehot = (zb[:, None] == lax.broadcasted_iota(jnp.int32, (1, 90), 1)).astype(
        jnp.float32
    )
    x0 = jnp.dot(onehot, emb_ref[...], preferred_element_type=jnp.float32)
    h = jnp.maximum(jnp.dot(x0, W1[...], preferred_element_type=jnp.float32) + b1[...], 0.0)
    x = jnp.dot(h, W2[...], preferred_element_type=jnp.float32) + b2[...]
    x_ref[...] = x
    xw_ref[...] = jnp.dot(x, emf[...], preferred_element_type=jnp.float32) + emb1[...]


def _enc_node(z, emb, W1, b1, W2, b2, emf, emb1):
    n = z.shape[0]
    grid = n // _BN
    z3 = z.reshape(grid, 1, _BN)
    return pl.pallas_call(
        _enc_node_body,
        grid=(grid,),
        in_specs=[
            pl.BlockSpec((1, 1, _BN), lambda i: (i, 0, 0)),
            _full(emb.shape), _full(W1.shape), _full(b1.shape),
            _full(W2.shape), _full(b2.shape), _full(emf.shape), _full(emb1.shape),
        ],
        out_specs=[
            pl.BlockSpec((_BN, H), lambda i: (i, 0)),
            pl.BlockSpec((_BN, H), lambda i: (i, 0)),
        ],
        out_shape=[
            jax.ShapeDtypeStruct((n, H), jnp.float32),
            jax.ShapeDtypeStruct((n, H), jnp.float32),
        ],
    )(z3, emb, W1, b1, W2, b2, emf, emb1)


def _enc_edge_body(ea_ref, W1, b1, W2, b2, out_ref):
    h = jnp.maximum(
        jnp.dot(ea_ref[...], W1[...], preferred_element_type=jnp.float32) + b1[...], 0.0
    )
    out_ref[...] = jnp.dot(h, W2[...], preferred_element_type=jnp.float32) + b2[...]


def _enc_edge(edge_attr, W1, b1, W2, b2):
    e, nb = edge_attr.shape
    grid = e // _BE
    return pl.pallas_call(
        _enc_edge_body,
        grid=(grid,),
        in_specs=[
            pl.BlockSpec((_BE, nb), lambda i: (i, 0)),
            _full(W1.shape), _full(b1.shape), _full(W2.shape), _full(b2.shape),
        ],
        out_specs=pl.BlockSpec((_BE, H), lambda i: (i, 0)),
        out_shape=jax.ShapeDtypeStruct((e, H), jnp.float32),
    )(edge_attr, W1, b1, W2, b2)


def _edge_step_body(g_ref, ea_ref, W1c, W2, b2, new_ref, sum_ref):
    ea = ea_ref[...]
    t = _ssp(g_ref[...] + jnp.dot(ea, W1c[...], preferred_element_type=jnp.float32))
    ea_new = jnp.dot(t, W2[...], preferred_element_type=jnp.float32) + b2[...]
    new_ref[...] = ea_new
    sum_ref[...] = ea + ea_new


def _edge_step(gathered, ea, W1c, W2, b2):
    e = ea.shape[0]
    grid = e // _BE
    return pl.pallas_call(
        _edge_step_body,
        grid=(grid,),
        in_specs=[
            pl.BlockSpec((_BE, H), lambda i: (i, 0)),
            pl.BlockSpec((_BE, H), lambda i: (i, 0)),
            _full(W1c.shape), _full(W2.shape), _full(b2.shape),
        ],
        out_specs=[
            pl.BlockSpec((_BE, H), lambda i: (i, 0)),
            pl.BlockSpec((_BE, H), lambda i: (i, 0)),
        ],
        out_shape=[
            jax.ShapeDtypeStruct((e, H), jnp.float32),
            jax.ShapeDtypeStruct((e, H), jnp.float32),
        ],
    )(gathered, ea, W1c, W2, b2)


def _node_step_body(x_ref, a0_ref, a1_ref, W1a, W1b, b1, W2, b2, emf, emb1,
                    x_out_ref, xw_ref):
    x = x_ref[...]
    agg = a0_ref[0, :, :] + a1_ref[0, :, :]
    t = _ssp(
        jnp.dot(x, W1a[...], preferred_element_type=jnp.float32)
        + jnp.dot(agg, W1b[...], preferred_element_type=jnp.float32)
        + b1[...]
    )
    x_out = x + jnp.dot(t, W2[...], preferred_element_type=jnp.float32) + b2[...]
    x_out_ref[...] = x_out
    xw_ref[...] = jnp.dot(x_out, emf[...], preferred_element_type=jnp.float32) + emb1[...]


def _node_step(x, agg2, W1a, W1b, b1, W2, b2, emf, emb1):
    n = x.shape[0]
    grid = n // _BN
    return pl.pallas_call(
        _node_step_body,
        grid=(grid,),
        in_specs=[
            pl.BlockSpec((_BN, H), lambda i: (i, 0)),
            pl.BlockSpec((1, _BN, H), lambda i: (0, i, 0)),
            pl.BlockSpec((1, _BN, H), lambda i: (1, i, 0)),
            _full(W1a.shape), _full(W1b.shape), _full(b1.shape),
            _full(W2.shape), _full(b2.shape), _full(emf.shape), _full(emb1.shape),
        ],
        out_specs=[
            pl.BlockSpec((_BN, H), lambda i: (i, 0)),
            pl.BlockSpec((_BN, H), lambda i: (i, 0)),
        ],
        out_shape=[
            jax.ShapeDtypeStruct((n, H), jnp.float32),
            jax.ShapeDtypeStruct((n, H), jnp.float32),
        ],
    )(x, agg2, agg2, W1a, W1b, b1, W2, b2, emf, emb1)


def _dec_body(x1, x2, x3, xe, batch_ref, pmW, pmb, pmg, pmbt,
              qmW, qmb, qmg, qmbt, recW, recb,
              recon_ref, pp_ref, pe_ref):
    i = pl.program_id(0)
    bat = batch_ref[0, 0, :]
    onehot = (bat[:, None] == lax.broadcasted_iota(jnp.int32, (1, 64), 1)).astype(
        jnp.float32
    )

    @pl.when(i == 0)
    def _():
        pp_ref[...] = jnp.zeros_like(pp_ref)
        pe_ref[...] = jnp.zeros_like(pe_ref)

    for k, xr in enumerate((x1, x2, x3)):
        xv = xr[...]
        recon_ref[k, :, :] = (
            jnp.dot(xv, recW[...], preferred_element_type=jnp.float32) + recb[...]
        )
        h = xv
        for l in range(3):
            h = _ssp(_ln(
                jnp.dot(h, pmW[l], preferred_element_type=jnp.float32) + pmb[l],
                pmg[l], pmbt[l]))
        pp_ref[k, :, :] += jnp.dot(onehot.T, h, preferred_element_type=jnp.float32)

    he = xe[...]
    for l in range(3):
        he = _ssp(_ln(
            jnp.dot(he, qmW[l], preferred_element_type=jnp.float32) + qmb[l],
            qmg[l], qmbt[l]))
    pe_ref[...] += jnp.dot(onehot.T, he, preferred_element_type=jnp.float32)


def _decoder(x1, x2, x3, enc_x, batch, pmW, pmb, pmg, pmbt, qmW, qmb, qmg, qmbt,
             recW, recb):
    n = batch.shape[0]
    grid = n // _BN
    b3 = batch.reshape(grid, 1, _BN)
    return pl.pallas_call(
        _dec_body,
        grid=(grid,),
        in_specs=[
            pl.BlockSpec((_BN, H), lambda i: (i, 0)),
            pl.BlockSpec((_BN, H), lambda i: (i, 0)),
            pl.BlockSpec((_BN, H), lambda i: (i, 0)),
            pl.BlockSpec((_BN, H), lambda i: (i, 0)),
            pl.BlockSpec((1, 1, _BN), lambda i: (i, 0, 0)),
            _full(pmW.shape), _full(pmb.shape), _full(pmg.shape), _full(pmbt.shape),
            _full(qmW.shape), _full(qmb.shape), _full(qmg.shape), _full(qmbt.shape),
            _full(recW.shape), _full(recb.shape),
        ],
        out_specs=[
            pl.BlockSpec((3, _BN, 3), lambda i: (0, i, 0)),
            pl.BlockSpec((3, 64, H), lambda i: (0, 0, 0)),
            pl.BlockSpec((64, H), lambda i: (0, 0)),
        ],
        out_shape=[
            jax.ShapeDtypeStruct((3, n, 3), jnp.float32),
            jax.ShapeDtypeStruct((3, 64, H), jnp.float32),
            jax.ShapeDtypeStruct((64, H), jnp.float32),
        ],
    )(x1, x2, x3, enc_x, b3, pmW, pmb, pmg, pmbt, qmW, qmb, qmg, qmbt, recW, recb)


def _target_body(pp_ref, pe_ref, pW, pb, eW, eb, out_ref):
    pe_term = jnp.dot(pe_ref[...], eW[...], preferred_element_type=jnp.float32) + eb[...]
    for k in range(3):
        tk = jnp.dot(pp_ref[k], pW[...], preferred_element_type=jnp.float32) + pb[...]
        out_ref[k, :] = (tk + pe_term)[:, 0]


def _target(pp, pe, pW, pb, eW, eb):
    return pl.pallas_call(
        _target_body,
        grid=(1,),
        in_specs=[
            _full(pp.shape), _full(pe.shape), _full(pW.shape),
            _full(pb.shape), _full(eW.shape), _full(eb.shape),
        ],
        out_specs=pl.BlockSpec((3, 64), lambda i: (0, 0)),
        out_shape=jax.ShapeDtypeStruct((3, 64), jnp.float32),
    )(pp, pe, pW, pb, eW, eb)


def kernel(z, edge_attr, edge_index, batch, params):
    p = params
    n_nodes = z.shape[0]

    em_W1 = p['em_W1']
    em_fold = em_W1[:H] + em_W1[H:2 * H]
    em_W1c = em_W1[2 * H:]
    nm_W1a = p['nm_W1'][:H]
    nm_W1b = p['nm_W1'][H:]

    row = edge_index[0]
    dst = edge_index[1]
    n_edges = row.shape[0]
    e2 = n_edges // 2
    row_h = (row[:e2], row[e2:])
    dst_h = (dst[:e2], dst[e2:])

    enc_x, xw = _enc_node(z, p['embed'], p['ne_W1'], p['ne_b1'], p['ne_W2'],
                          p['ne_b2'], em_fold, p['em_b1'])
    ea_h = [
        _enc_edge(edge_attr[:e2], p['ee_W1'], p['ee_b1'], p['ee_W2'], p['ee_b2']),
        _enc_edge(edge_attr[e2:], p['ee_W1'], p['ee_b1'], p['ee_W2'], p['ee_b2']),
    ]

    x = enc_x
    xs = []
    for _b in range(3):
        for _m in range(2):
            new_h = [None, None]
            part = None
            for h in range(2):
                g = _sc_gather(xw, row_h[h])
                ea_new, new_h[h] = _edge_step(g, ea_h[h], em_W1c,
                                              p['em_W2'], p['em_b2'])
                part = _sc_scatter(ea_new, dst_h[h], n_nodes, init=part)
            ea_h = new_h
            x, xw = _node_step(x, part, nm_W1a, nm_W1b, p['nm_b1'],
                               p['nm_W2'], p['nm_b2'], em_fold, p['em_b1'])
        xs.append(x)

    x_list = jnp.stack(xs, axis=0)

    pmW = jnp.stack(p['pm_W']); pmb = jnp.stack(p['pm_b'])
    pmg = jnp.stack(p['pm_g']); pmbt = jnp.stack(p['pm_beta'])
    qmW = jnp.stack(p['qm_W']); qmb = jnp.stack(p['qm_b'])
    qmg = jnp.stack(p['qm_g']); qmbt = jnp.stack(p['qm_beta'])

    recon, pooled_proc, pooled_enc = _decoder(
        xs[0], xs[1], xs[2], enc_x, batch,
        pmW, pmb, pmg, pmbt, qmW, qmb, qmg, qmbt, p['rec_W'], p['rec_b'])

    target = _target(pooled_proc, pooled_enc, p['pW'], p['pb'], p['eW'], p['eb'])

    return (target, recon, x_list)
